# baseline (speedup 1.0000x reference)
import functools

import jax
import jax.numpy as jnp
from jax import lax
from jax.experimental import pallas as pl
from jax.experimental.pallas import tpu as pltpu
from jax.experimental.pallas import tpu_sc as plsc

_NUM_CORES = 2
_NUM_SUBCORES = 16
_CHUNK = 128


def _gather_sc(idx2d, table, n_chunks, b_per_w, d):
    nw = _NUM_CORES * _NUM_SUBCORES
    batch = nw * b_per_w

    @functools.partial(
        pl.kernel,
        mesh=plsc.VectorSubcoreMesh(core_axis_name="c", subcore_axis_name="s"),
        out_type=jax.ShapeDtypeStruct((batch, d), jnp.float32),
        scratch_types=[
            pltpu.VMEM((n_chunks, _CHUNK), jnp.int32),
            pltpu.VMEM((b_per_w, d), jnp.float32),
            pltpu.SemaphoreType.DMA,
        ],
        compiler_params=pltpu.CompilerParams(use_tc_tiling_on_sc=False),
    )
    def gather_kernel(idx_hbm, table_hbm, out_hbm, idx_v, rows_v, sem):
        wid = lax.axis_index("s") * _NUM_CORES + lax.axis_index("c")
        pltpu.sync_copy(idx_hbm.at[pl.ds(wid * n_chunks, n_chunks)], idx_v)
        copies = [
            pltpu.async_copy(
                table_hbm.at[idx_v.at[j]],
                rows_v.at[pl.ds(j * _CHUNK, _CHUNK)],
                sem,
            )
            for j in range(n_chunks)
        ]
        for c in copies:
            c.wait()
        pltpu.sync_copy(rows_v, out_hbm.at[pl.ds(wid * b_per_w, b_per_w)])

    return gather_kernel(idx2d, table)


def kernel(index, table):
    b = index.shape[0]
    d = table.shape[1]
    nw = _NUM_CORES * _NUM_SUBCORES
    b_per_w = b // nw
    n_chunks = b_per_w // _CHUNK
    idx2d = index.astype(jnp.int32).reshape(nw * n_chunks, _CHUNK)
    out = _gather_sc(idx2d, table, n_chunks, b_per_w, d)
    return out.reshape(b, d, 1, 1)


# speedup vs baseline: 4.3911x; 4.3911x over previous
import functools

import jax
import jax.numpy as jnp
from jax import lax
from jax.experimental import pallas as pl
from jax.experimental.pallas import tpu as pltpu
from jax.experimental.pallas import tpu_sc as plsc

_NUM_CORES = 2
_NUM_SUBCORES = 16
_LANES = 16
_TILE_W = 128
_BATCH = 8


def _gather_sc(idx, table_t, b, d):
    nw = _NUM_CORES * _NUM_SUBCORES
    b_per_w = b // nw
    n_batches = b_per_w // _BATCH

    @functools.partial(
        pl.kernel,
        mesh=plsc.VectorSubcoreMesh(core_axis_name="c", subcore_axis_name="s"),
        out_type=jax.ShapeDtypeStruct((d * b,), jnp.float32),
        scratch_types=[
            pltpu.VMEM((b_per_w,), jnp.int32),
            pltpu.VMEM((2 * _BATCH, d, _TILE_W), jnp.float32),
            pltpu.VMEM((d * b_per_w,), jnp.float32),
            pltpu.SemaphoreType.DMA,
        ],
        compiler_params=pltpu.CompilerParams(
            use_tc_tiling_on_sc=True, needs_layout_passes=False
        ),
    )
    def gather_kernel(idx_hbm, table_hbm, out_hbm, idx_v, buf_v, vals_v, sem):
        wid = lax.axis_index("s") * _NUM_CORES + lax.axis_index("c")
        base = wid * b_per_w
        pltpu.sync_copy(idx_hbm.at[pl.ds(base, b_per_w)], idx_v)

        lanes = lax.iota(jnp.int32, _LANES)

        def splat(x):
            return jnp.full((_LANES,), x, jnp.int32)

        def fire(g, half):
            for k in range(_BATCH):
                ivec = plsc.load_gather(idx_v, [splat(g * _BATCH + k)])
                col = (ivec[0] >> 7) * _TILE_W
                col = pl.multiple_of(col, _TILE_W)
                pltpu.async_copy(
                    table_hbm.at[:, pl.ds(col, _TILE_W)],
                    buf_v.at[half * _BATCH + k],
                    sem,
                )

        def drain(half):
            for k in range(_BATCH):
                pltpu.make_async_copy(
                    table_hbm.at[:, pl.ds(0, _TILE_W)],
                    buf_v.at[half * _BATCH + k],
                    sem,
                ).wait()

        def extract(g, half):
            for k in range(_BATCH):
                li = g * _BATCH + k
                ivec = plsc.load_gather(idx_v, [splat(li)])
                lane = ivec & (_TILE_W - 1)
                slot_v = splat(half * _BATCH + k)
                for h in range(d // _LANES):
                    evec = lanes + h * _LANES
                    v = plsc.load_gather(buf_v, [slot_v, evec, lane])
                    plsc.store_scatter(vals_v, [evec * b_per_w + li], v)

        fire(0, 0)

        def pair_body(gp, carry):
            ga = gp * 2
            fire(ga + 1, 1)
            drain(0)
            extract(ga, 0)
            fire(ga + 2, 0)
            drain(1)
            extract(ga + 1, 1)
            return carry

        lax.fori_loop(0, n_batches // 2 - 1, pair_body, jnp.int32(0))

        fire(n_batches - 1, 1)
        drain(0)
        extract(n_batches - 2, 0)
        drain(1)
        extract(n_batches - 1, 1)

        for e in range(d):
            pltpu.sync_copy(
                vals_v.at[pl.ds(e * b_per_w, b_per_w)],
                out_hbm.at[pl.ds(e * b + base, b_per_w)],
            )

    return gather_kernel(idx, table_t)


def kernel(index, table):
    b = index.shape[0]
    d = table.shape[1]
    idx = index.astype(jnp.int32)
    out_flat = _gather_sc(idx, table.T, b, d)
    return out_flat.reshape(d, b).T.reshape(b, d, 1, 1)


# speedup vs baseline: 4.4259x; 1.0079x over previous
import functools

import jax
import jax.numpy as jnp
from jax import lax
from jax.experimental import pallas as pl
from jax.experimental.pallas import tpu as pltpu
from jax.experimental.pallas import tpu_sc as plsc

_NUM_CORES = 2
_NUM_SUBCORES = 16
_LANES = 16
_TILE_W = 128
_BATCH = 8


def _gather_sc(idx, table_t, b, d):
    nw = _NUM_CORES * _NUM_SUBCORES
    b_per_w = b // nw
    n_batches = b_per_w // _BATCH

    @functools.partial(
        pl.kernel,
        mesh=plsc.VectorSubcoreMesh(core_axis_name="c", subcore_axis_name="s"),
        out_type=jax.ShapeDtypeStruct((d * b,), jnp.float32),
        scratch_types=[
            pltpu.VMEM((b_per_w,), jnp.int32),
            pltpu.VMEM((2 * _BATCH, d, _TILE_W), jnp.float32),
            pltpu.VMEM((d * b_per_w,), jnp.float32),
            pltpu.SemaphoreType.DMA,
        ],
        compiler_params=pltpu.CompilerParams(
            use_tc_tiling_on_sc=True, needs_layout_passes=False
        ),
    )
    def gather_kernel(idx_hbm, table_hbm, out_hbm, idx_v, buf_v, vals_v, sem):
        wid = lax.axis_index("s") * _NUM_CORES + lax.axis_index("c")
        base = wid * b_per_w
        pltpu.sync_copy(idx_hbm.at[pl.ds(base, b_per_w)], idx_v)

        lanes = lax.iota(jnp.int32, _LANES)

        def splat(x):
            return jnp.full((_LANES,), x, jnp.int32)

        def load_pair(ga):
            return idx_v[pl.ds(ga * _BATCH, 2 * _BATCH)]

        def fire(vec, off, half):
            for k in range(_BATCH):
                col = (vec[off + k] >> 7) * _TILE_W
                col = pl.multiple_of(col, _TILE_W)
                pltpu.async_copy(
                    table_hbm.at[:, pl.ds(col, _TILE_W)],
                    buf_v.at[half * _BATCH + k],
                    sem,
                )

        def drain(half):
            for k in range(_BATCH):
                pltpu.make_async_copy(
                    table_hbm.at[:, pl.ds(0, _TILE_W)],
                    buf_v.at[half * _BATCH + k],
                    sem,
                ).wait()

        def extract(vec, off, g, half):
            for k in range(_BATCH):
                li = g * _BATCH + k
                lane = splat(vec[off + k] & (_TILE_W - 1))
                slot_v = splat(half * _BATCH + k)
                for h in range(d // _LANES):
                    evec = lanes + h * _LANES
                    v = plsc.load_gather(buf_v, [slot_v, evec, lane])
                    plsc.store_scatter(vals_v, [evec * b_per_w + li], v)

        vec0 = load_pair(0)
        fire(vec0, 0, 0)

        def pair_body(gp, carry):
            ga = gp * 2
            vec = load_pair(ga)
            vec_n = load_pair(ga + 2)
            fire(vec, _BATCH, 1)
            drain(0)
            extract(vec, 0, ga, 0)
            fire(vec_n, 0, 0)
            drain(1)
            extract(vec, _BATCH, ga + 1, 1)
            return carry

        lax.fori_loop(0, n_batches // 2 - 1, pair_body, jnp.int32(0))

        ga = n_batches - 2
        vec = load_pair(ga)
        fire(vec, _BATCH, 1)
        drain(0)
        extract(vec, 0, ga, 0)
        drain(1)
        extract(vec, _BATCH, ga + 1, 1)

        for e in range(d):
            pltpu.sync_copy(
                vals_v.at[pl.ds(e * b_per_w, b_per_w)],
                out_hbm.at[pl.ds(e * b + base, b_per_w)],
            )

    return gather_kernel(idx, table_t)


def kernel(index, table):
    b = index.shape[0]
    d = table.shape[1]
    idx = index.astype(jnp.int32)
    out_flat = _gather_sc(idx, table.T, b, d)
    return out_flat.reshape(d, b).T.reshape(b, d, 1, 1)
